# baseline (speedup 1.0000x reference)
import functools

import jax
import jax.numpy as jnp
from jax import lax
from jax.experimental import pallas as pl
from jax.experimental.pallas import tpu as pltpu
from jax.experimental.pallas import tpu_sc as plsc

NC = 2
NS = 16
NW = NC * NS

SUB = 128
SUBS = 4
CHUNK = SUB * SUBS


def _emb_kernel(B, D, b_per_w, n_chunks,
                idx_hbm, table_hbm, out_hbm,
                idx_v, rows_v, sem):
    wid = lax.axis_index("s") * NC + lax.axis_index("c")
    wbase = wid * b_per_w

    pltpu.sync_copy(idx_hbm.at[pl.ds(wbase, b_per_w)], idx_v)

    def chunk_body(g):
        off = g * CHUNK
        for j in range(SUBS):
            pltpu.async_copy(
                table_hbm.at[idx_v.at[pl.ds(off + j * SUB, SUB)]],
                rows_v.at[pl.ds(j * SUB, SUB)],
                sem,
            )
        pltpu.make_async_copy(table_hbm.at[pl.ds(0, CHUNK)], rows_v, sem).wait()
        pltpu.sync_copy(rows_v, out_hbm.at[pl.ds(wbase + off, CHUNK)])

    pl.loop(0, n_chunks)(chunk_body)


def _make_emb(B, D):
    assert B % (NW * CHUNK) == 0
    b_per_w = B // NW
    n_chunks = b_per_w // CHUNK
    mesh = plsc.VectorSubcoreMesh(core_axis_name="c", subcore_axis_name="s")
    return pl.kernel(
        functools.partial(_emb_kernel, B, D, b_per_w, n_chunks),
        out_type=jax.ShapeDtypeStruct((B, D), jnp.float32),
        mesh=mesh,
        scratch_types=[
            pltpu.VMEM((b_per_w,), jnp.int32),
            pltpu.VMEM((CHUNK, D), jnp.float32),
            pltpu.SemaphoreType.DMA,
        ],
        compiler_params=pltpu.CompilerParams(use_tc_tiling_on_sc=False),
    )


@jax.jit
def kernel(input_tensor, W):
    B = input_tensor.size
    D = W.shape[1]
    idx_flat = input_tensor.reshape(B).astype(jnp.int32)
    out = _make_emb(B, D)(idx_flat, W)
    return out.reshape(*input_tensor.shape, D)
